# Initial kernel scaffold; baseline (speedup 1.0000x reference)
import jax
import jax.numpy as jnp
from jax.experimental import pallas as pl


def kernel(x, edge_index, W1, b1, W2, b2):
    raise NotImplementedError("write your pallas kernel here")


# speedup vs baseline: 20.2484x; 20.2484x over previous
import functools

import jax
import jax.numpy as jnp
from jax import lax
from jax.experimental import pallas as pl
from jax.experimental.pallas import tpu as pltpu
from jax.experimental.pallas import tpu_sc as plsc

NC = 2
NS = 16
NW = NC * NS
K = 128


@functools.lru_cache(maxsize=None)
def _make_seg_sum(N, E, C, gather):
    PER_W = E // NW
    FULL = PER_W // K
    TAIL = PER_W % K
    NP = -(-N // (NS * 8)) * (NS * 8)
    ROWS_T = NP // NS
    mesh = plsc.VectorSubcoreMesh(core_axis_name="c", subcore_axis_name="s")

    scratch = [
        pltpu.VMEM_SHARED((NP, C), jnp.float32),
        pltpu.VMEM((1, K), jnp.int32),
        pltpu.VMEM((1, K), jnp.int32),
        pltpu.VMEM((K, C), jnp.float32),
        pltpu.SemaphoreType.DMA,
    ]
    if TAIL:
        scratch += [
            pltpu.VMEM((1, TAIL), jnp.int32),
            pltpu.VMEM((1, TAIL), jnp.int32),
            pltpu.VMEM((TAIL, C), jnp.float32),
        ]

    @functools.partial(
        pl.kernel,
        out_type=jax.ShapeDtypeStruct((NC, NP, C), jnp.float32),
        mesh=mesh,
        scratch_types=scratch,
        compiler_params=pltpu.CompilerParams(use_tc_tiling_on_sc=False),
    )
    def k(*refs):
        if gather:
            table, src, dst, zeros, out = refs[:5]
            refs = refs[5:]
        else:
            ones, dst, zeros, out = refs[:4]
            refs = refs[4:]
        acc, srcb, dstb, rowsb, sem = refs[:5]
        if TAIL:
            srct, dstt, rowst = refs[5:]

        cid = lax.axis_index("c")
        sid = lax.axis_index("s")
        base = (cid * NS + sid) * PER_W

        pltpu.sync_copy(zeros, acc.at[pl.ds(sid * ROWS_T, ROWS_T)])
        if not gather:
            pltpu.sync_copy(ones, rowsb)
            if TAIL:
                pltpu.sync_copy(ones.at[pl.ds(0, TAIL)], rowst)
        plsc.subcore_barrier()

        def body(i, carry):
            off = base + i * K
            pltpu.sync_copy(dst.at[pl.ds(off, K)], dstb.at[0])
            if gather:
                pltpu.sync_copy(src.at[pl.ds(off, K)], srcb.at[0])
                pltpu.async_copy(table.at[srcb.at[0]], rowsb, sem).wait()
            pltpu.sync_copy(rowsb, acc.at[dstb.at[0]], add=True)
            return carry

        lax.fori_loop(0, FULL, body, 0)

        if TAIL:
            off = base + FULL * K
            pltpu.sync_copy(dst.at[pl.ds(off, TAIL)], dstt.at[0])
            if gather:
                pltpu.sync_copy(src.at[pl.ds(off, TAIL)], srct.at[0])
                pltpu.async_copy(table.at[srct.at[0]], rowst, sem).wait()
            pltpu.sync_copy(rowst, acc.at[dstt.at[0]], add=True)

        plsc.subcore_barrier()
        pltpu.sync_copy(
            acc.at[pl.ds(sid * ROWS_T, ROWS_T)],
            out.at[cid, pl.ds(sid * ROWS_T, ROWS_T)],
        )

    return k


def _grid_specs(N, shapes):
    GB = 10
    RB = N // GB
    specs = []
    for shape, row_dim in shapes:
        if row_dim is None:
            specs.append(pl.BlockSpec(shape, lambda i: tuple(0 for _ in shape)))
        else:
            blk = tuple(RB if d == row_dim else s for d, s in enumerate(shape))
            specs.append(
                pl.BlockSpec(
                    blk,
                    functools.partial(
                        lambda i, rd=row_dim, r=len(shape): tuple(
                            i if d == rd else 0 for d in range(r)
                        )
                    ),
                )
            )
    return GB, specs


@functools.lru_cache(maxsize=None)
def _make_stage1(N, IN, C1, CD):
    GB = 10
    RB = N // GB

    def body(degp, xr, w, hs1, dinv8):
        d = degp[0] + degp[1] + 1.0
        di = lax.rsqrt(d)
        dinv8[...] = di
        hs1[...] = (
            jnp.dot(xr[...], w[...], preferred_element_type=jnp.float32)
            * di[:, 0:1]
        )

    return pl.pallas_call(
        body,
        grid=(GB,),
        in_specs=[
            pl.BlockSpec((2, RB, CD), lambda i: (0, i, 0)),
            pl.BlockSpec((RB, IN), lambda i: (i, 0)),
            pl.BlockSpec((IN, C1), lambda i: (0, 0)),
        ],
        out_specs=[
            pl.BlockSpec((RB, C1), lambda i: (i, 0)),
            pl.BlockSpec((RB, CD), lambda i: (i, 0)),
        ],
        out_shape=[
            jax.ShapeDtypeStruct((N, C1), jnp.float32),
            jax.ShapeDtypeStruct((N, CD), jnp.float32),
        ],
    )


@functools.lru_cache(maxsize=None)
def _make_stage2(N, C1, C2, CD):
    GB = 10
    RB = N // GB

    def body(acc1, hs1, dinv8, b1, w2, hs2):
        di = dinv8[:, 0:1]
        h = jnp.maximum((acc1[0] + acc1[1] + hs1[...]) * di + b1[...], 0.0)
        hs2[...] = (
            jnp.dot(h, w2[...], preferred_element_type=jnp.float32) * di
        )

    return pl.pallas_call(
        body,
        grid=(GB,),
        in_specs=[
            pl.BlockSpec((2, RB, C1), lambda i: (0, i, 0)),
            pl.BlockSpec((RB, C1), lambda i: (i, 0)),
            pl.BlockSpec((RB, CD), lambda i: (i, 0)),
            pl.BlockSpec((1, C1), lambda i: (0, 0)),
            pl.BlockSpec((C1, C2), lambda i: (0, 0)),
        ],
        out_specs=pl.BlockSpec((RB, C2), lambda i: (i, 0)),
        out_shape=jax.ShapeDtypeStruct((N, C2), jnp.float32),
    )


@functools.lru_cache(maxsize=None)
def _make_stage3(N, C2, CD):
    GB = 10
    RB = N // GB

    def body(acc2, hs2, dinv8, b2, z):
        di = dinv8[:, 0:1]
        z[...] = (acc2[0] + acc2[1] + hs2[...]) * di + b2[...]

    return pl.pallas_call(
        body,
        grid=(GB,),
        in_specs=[
            pl.BlockSpec((2, RB, C2), lambda i: (0, i, 0)),
            pl.BlockSpec((RB, C2), lambda i: (i, 0)),
            pl.BlockSpec((RB, CD), lambda i: (i, 0)),
            pl.BlockSpec((1, C2), lambda i: (0, 0)),
        ],
        out_specs=pl.BlockSpec((RB, C2), lambda i: (i, 0)),
        out_shape=jax.ShapeDtypeStruct((N, C2), jnp.float32),
    )


def kernel(x, edge_index, W1, b1, W2, b2):
    N, IN = x.shape
    HID = W1.shape[1]
    OUT = W2.shape[1]
    E = edge_index.shape[1]
    assert E % NW == 0 and N % NS == 0

    C1 = 32
    C2 = 16
    CD = 8

    src = edge_index[0].astype(jnp.int32)
    dst = edge_index[1].astype(jnp.int32)
    f32 = jnp.float32
    W1p = jnp.zeros((IN, C1), f32).at[:, :HID].set(W1)
    b1p = jnp.zeros((1, C1), f32).at[0, :HID].set(b1)
    W2p = jnp.zeros((C1, C2), f32).at[:HID, :OUT].set(W2)
    b2p = jnp.zeros((1, C2), f32).at[0, :OUT].set(b2)
    ROWS_T = (-(-N // (NS * 8)) * (NS * 8)) // NS
    zerosD = jnp.zeros((ROWS_T, CD), f32)
    zeros1 = jnp.zeros((ROWS_T, C1), f32)
    zeros2 = jnp.zeros((ROWS_T, C2), f32)
    onesD = jnp.ones((K, CD), f32)

    deg_parts = _make_seg_sum(N, E, CD, False)(onesD, dst, zerosD)[:, :N]
    hs1, dinv8 = _make_stage1(N, IN, C1, CD)(deg_parts, x, W1p)
    acc1 = _make_seg_sum(N, E, C1, True)(hs1, src, dst, zeros1)[:, :N]
    hs2 = _make_stage2(N, C1, C2, CD)(acc1, hs1, dinv8, b1p, W2p)
    acc2 = _make_seg_sum(N, E, C2, True)(hs2, src, dst, zeros2)[:, :N]
    zf = _make_stage3(N, C2, CD)(acc2, hs2, dinv8, b2p)
    return zf[:, :OUT]


# speedup vs baseline: 35.8953x; 1.7727x over previous
import functools

import jax
import jax.numpy as jnp
from jax import lax
from jax.experimental import pallas as pl
from jax.experimental.pallas import tpu as pltpu
from jax.experimental.pallas import tpu_sc as plsc

NC = 2
NS = 16
NW = NC * NS
K = 128


def _pad_rows(N):
    return -(-(N + 1) // (NS * 8)) * (NS * 8)


@functools.lru_cache(maxsize=None)
def _make_seg_sum(N, EP, C, gather):
    CPT = EP // (NW * K)
    NB = 4
    D = 3
    NP = _pad_rows(N)
    ROWS_T = NP // NS
    assert CPT >= D + 2
    mesh = plsc.VectorSubcoreMesh(core_axis_name="c", subcore_axis_name="s")

    scratch = [
        pltpu.VMEM_SHARED((NP, C), jnp.float32),
        pltpu.VMEM((CPT, K), jnp.int32),
        pltpu.VMEM((NB, K, C), jnp.float32),
        pltpu.SemaphoreType.DMA,
    ]
    if gather:
        scratch += [
            pltpu.VMEM((CPT, K), jnp.int32),
            pltpu.SemaphoreType.DMA,
        ]

    @functools.partial(
        pl.kernel,
        out_type=jax.ShapeDtypeStruct((NC, NP, C), jnp.float32),
        mesh=mesh,
        scratch_types=scratch,
        compiler_params=pltpu.CompilerParams(use_tc_tiling_on_sc=False),
    )
    def k(*refs):
        if gather:
            table, src2, dst2, zeros, out, acc, dstb, rows, ssem, srcb, gsem = refs
        else:
            ones, dst2, zeros, out, acc, dstb, rows, ssem = refs

        cid = lax.axis_index("c")
        sid = lax.axis_index("s")
        w = cid * NS + sid

        pltpu.sync_copy(dst2.at[pl.ds(w * CPT, CPT)], dstb)

        def s_issue(j, b):
            pltpu.async_copy(rows.at[b], acc.at[dstb.at[j]], ssem, add=True)

        def s_wait(j, b):
            pltpu.make_async_copy(rows.at[b], acc.at[dstb.at[j]], ssem).wait()

        if gather:
            pltpu.sync_copy(src2.at[pl.ds(w * CPT, CPT)], srcb)

            def g_issue(j, b):
                pltpu.async_copy(table.at[srcb.at[j]], rows.at[b], gsem)

            def g_wait(j, b):
                pltpu.make_async_copy(
                    table.at[srcb.at[j]], rows.at[b], gsem
                ).wait()

            for j in range(D):
                g_issue(j, j)
        else:
            pltpu.sync_copy(ones, rows.at[0])

        pltpu.sync_copy(zeros, acc.at[pl.ds(sid * ROWS_T, ROWS_T)])
        plsc.subcore_barrier()

        if gather:
            g_wait(0, 0)
            s_issue(0, 0)
            g_issue(D, D % NB)

            def body(i, carry):
                b = lax.rem(i, NB)
                g_wait(i, b)
                s_issue(i, b)
                s_wait(i - 1, lax.rem(i - 1, NB))
                g_issue(i + D, lax.rem(i + D, NB))
                return carry

            lax.fori_loop(1, CPT - D, body, 0)

            for i in range(CPT - D, CPT):
                g_wait(i, i % NB)
                s_issue(i, i % NB)
            for i in range(CPT - D - 1, CPT):
                s_wait(i, i % NB)
        else:
            F = 8
            for j in range(F):
                s_issue(j, 0)

            def body(i, carry):
                s_wait(i - F, 0)
                s_issue(i, 0)
                return carry

            lax.fori_loop(F, CPT, body, 0)
            for i in range(CPT - F, CPT):
                s_wait(i, 0)

        plsc.subcore_barrier()
        pltpu.sync_copy(
            acc.at[pl.ds(sid * ROWS_T, ROWS_T)],
            out.at[cid, pl.ds(sid * ROWS_T, ROWS_T)],
        )

    return k


@functools.lru_cache(maxsize=None)
def _make_stage1(N, IN, C1, CD):
    GB = 10
    RB = N // GB

    def body(degp, xr, w, hs1, dinv8):
        d = degp[0] + degp[1] + 1.0
        di = lax.rsqrt(d)
        dinv8[...] = di
        hs1[...] = (
            jnp.dot(xr[...], w[...], preferred_element_type=jnp.float32)
            * di[:, 0:1]
        )

    return pl.pallas_call(
        body,
        grid=(GB,),
        in_specs=[
            pl.BlockSpec((2, RB, CD), lambda i: (0, i, 0)),
            pl.BlockSpec((RB, IN), lambda i: (i, 0)),
            pl.BlockSpec((IN, C1), lambda i: (0, 0)),
        ],
        out_specs=[
            pl.BlockSpec((RB, C1), lambda i: (i, 0)),
            pl.BlockSpec((RB, CD), lambda i: (i, 0)),
        ],
        out_shape=[
            jax.ShapeDtypeStruct((N, C1), jnp.float32),
            jax.ShapeDtypeStruct((N, CD), jnp.float32),
        ],
    )


@functools.lru_cache(maxsize=None)
def _make_stage2(N, C1, C2, CD):
    GB = 10
    RB = N // GB

    def body(acc1, hs1, dinv8, b1, w2, hs2):
        di = dinv8[:, 0:1]
        h = jnp.maximum((acc1[0] + acc1[1] + hs1[...]) * di + b1[...], 0.0)
        hs2[...] = (
            jnp.dot(h, w2[...], preferred_element_type=jnp.float32) * di
        )

    return pl.pallas_call(
        body,
        grid=(GB,),
        in_specs=[
            pl.BlockSpec((2, RB, C1), lambda i: (0, i, 0)),
            pl.BlockSpec((RB, C1), lambda i: (i, 0)),
            pl.BlockSpec((RB, CD), lambda i: (i, 0)),
            pl.BlockSpec((1, C1), lambda i: (0, 0)),
            pl.BlockSpec((C1, C2), lambda i: (0, 0)),
        ],
        out_specs=pl.BlockSpec((RB, C2), lambda i: (i, 0)),
        out_shape=jax.ShapeDtypeStruct((N, C2), jnp.float32),
    )


@functools.lru_cache(maxsize=None)
def _make_stage3(N, C2, CD):
    GB = 10
    RB = N // GB

    def body(acc2, hs2, dinv8, b2, z):
        di = dinv8[:, 0:1]
        z[...] = (acc2[0] + acc2[1] + hs2[...]) * di + b2[...]

    return pl.pallas_call(
        body,
        grid=(GB,),
        in_specs=[
            pl.BlockSpec((2, RB, C2), lambda i: (0, i, 0)),
            pl.BlockSpec((RB, C2), lambda i: (i, 0)),
            pl.BlockSpec((RB, CD), lambda i: (i, 0)),
            pl.BlockSpec((1, C2), lambda i: (0, 0)),
        ],
        out_specs=pl.BlockSpec((RB, C2), lambda i: (i, 0)),
        out_shape=jax.ShapeDtypeStruct((N, C2), jnp.float32),
    )


def kernel(x, edge_index, W1, b1, W2, b2):
    N, IN = x.shape
    HID = W1.shape[1]
    OUT = W2.shape[1]
    E = edge_index.shape[1]
    assert N % NS == 0

    C1 = 32
    C2 = 16
    CD = 8

    f32 = jnp.float32
    src = edge_index[0].astype(jnp.int32)
    dst = edge_index[1].astype(jnp.int32)
    EP = -(-E // (NW * K)) * (NW * K)
    if EP != E:
        pad = EP - E
        src = jnp.concatenate([src, jnp.zeros((pad,), jnp.int32)])
        dst = jnp.concatenate([dst, jnp.full((pad,), N, jnp.int32)])
    src2 = src.reshape(EP // K, K)
    dst2 = dst.reshape(EP // K, K)

    W1p = jnp.zeros((IN, C1), f32).at[:, :HID].set(W1)
    b1p = jnp.zeros((1, C1), f32).at[0, :HID].set(b1)
    W2p = jnp.zeros((C1, C2), f32).at[:HID, :OUT].set(W2)
    b2p = jnp.zeros((1, C2), f32).at[0, :OUT].set(b2)
    ROWS_T = _pad_rows(N) // NS
    zerosD = jnp.zeros((ROWS_T, CD), f32)
    zeros1 = jnp.zeros((ROWS_T, C1), f32)
    zeros2 = jnp.zeros((ROWS_T, C2), f32)
    onesD = jnp.ones((K, CD), f32)

    deg_parts = _make_seg_sum(N, EP, CD, False)(onesD, dst2, zerosD)[:, :N]
    hs1, dinv8 = _make_stage1(N, IN, C1, CD)(deg_parts, x, W1p)
    acc1 = _make_seg_sum(N, EP, C1, True)(hs1, src2, dst2, zeros1)[:, :N]
    hs2 = _make_stage2(N, C1, C2, CD)(acc1, hs1, dinv8, b1p, W2p)
    acc2 = _make_seg_sum(N, EP, C2, True)(hs2, src2, dst2, zeros2)[:, :N]
    zf = _make_stage3(N, C2, CD)(acc2, hs2, dinv8, b2p)
    return zf[:, :OUT]


# speedup vs baseline: 38.3733x; 1.0690x over previous
import functools

import jax
import jax.numpy as jnp
from jax import lax
from jax.experimental import pallas as pl
from jax.experimental.pallas import tpu as pltpu
from jax.experimental.pallas import tpu_sc as plsc

NC = 2
NS = 16
NW = NC * NS
K = 128


def _pad_rows(N):
    return -(-(N + 1) // (NS * 8)) * (NS * 8)


@functools.lru_cache(maxsize=None)
def _make_seg_sum(N, EP, C, gather):
    CPT = EP // (NW * K)
    NB = 4
    D = 3
    NP = _pad_rows(N)
    ROWS_T = NP // NS
    assert CPT >= D + 2
    mesh = plsc.VectorSubcoreMesh(core_axis_name="c", subcore_axis_name="s")

    scratch = [
        pltpu.VMEM_SHARED((NP, C), jnp.float32),
        pltpu.VMEM((CPT, K), jnp.int32),
        pltpu.VMEM((NB, K, C), jnp.float32),
        pltpu.SemaphoreType.DMA,
    ]
    if gather:
        scratch += [
            pltpu.VMEM((CPT, K), jnp.int32),
            pltpu.SemaphoreType.DMA,
        ]

    @functools.partial(
        pl.kernel,
        out_type=jax.ShapeDtypeStruct((NC, NP, C), jnp.float32),
        mesh=mesh,
        scratch_types=scratch,
        compiler_params=pltpu.CompilerParams(use_tc_tiling_on_sc=False),
    )
    def k(*refs):
        if gather:
            table, src2, dst2, zeros, out, acc, dstb, rows, ssem, srcb, gsem = refs
        else:
            ones, dst2, zeros, out, acc, dstb, rows, ssem = refs

        cid = lax.axis_index("c")
        sid = lax.axis_index("s")
        w = cid * NS + sid

        pltpu.sync_copy(dst2.at[pl.ds(w * CPT, CPT)], dstb)

        def s_issue(j, b):
            pltpu.async_copy(rows.at[b], acc.at[dstb.at[j]], ssem, add=True)

        def s_wait(j, b):
            pltpu.make_async_copy(rows.at[b], acc.at[dstb.at[j]], ssem).wait()

        if gather:
            pltpu.sync_copy(src2.at[pl.ds(w * CPT, CPT)], srcb)

            def g_issue(j, b):
                pltpu.async_copy(table.at[srcb.at[j]], rows.at[b], gsem)

            def g_wait(j, b):
                pltpu.make_async_copy(
                    table.at[srcb.at[j]], rows.at[b], gsem
                ).wait()

            for j in range(D):
                g_issue(j, j)
        else:
            pltpu.sync_copy(ones, rows.at[0])

        pltpu.sync_copy(zeros, acc.at[pl.ds(sid * ROWS_T, ROWS_T)])
        plsc.subcore_barrier()

        if gather:
            g_wait(0, 0)
            s_issue(0, 0)
            g_issue(D, D % NB)

            def body(i, carry):
                b = lax.rem(i, NB)
                g_wait(i, b)
                s_issue(i, b)
                s_wait(i - 1, lax.rem(i - 1, NB))
                g_issue(i + D, lax.rem(i + D, NB))
                return carry

            lax.fori_loop(1, CPT - D, body, 0)

            for i in range(CPT - D, CPT):
                g_wait(i, i % NB)
                s_issue(i, i % NB)
            for i in range(CPT - D - 1, CPT):
                s_wait(i, i % NB)
        else:
            F = 8
            for j in range(F):
                s_issue(j, 0)

            def body(i, carry):
                s_wait(i - F, 0)
                s_issue(i, 0)
                return carry

            lax.fori_loop(F, CPT, body, 0)
            for i in range(CPT - F, CPT):
                s_wait(i, 0)

        plsc.subcore_barrier()
        pltpu.sync_copy(
            acc.at[pl.ds(sid * ROWS_T, ROWS_T)],
            out.at[cid, pl.ds(sid * ROWS_T, ROWS_T)],
        )

    return k


@functools.lru_cache(maxsize=None)
def _make_mm(N, IN, C1):
    GB = 10
    RB = N // GB

    def body(xr, w, xw):
        xw[...] = jnp.dot(xr[...], w[...], preferred_element_type=jnp.float32)

    return pl.pallas_call(
        body,
        grid=(GB,),
        in_specs=[
            pl.BlockSpec((RB, IN), lambda i: (i, 0)),
            pl.BlockSpec((IN, C1), lambda i: (0, 0)),
        ],
        out_specs=pl.BlockSpec((RB, C1), lambda i: (i, 0)),
        out_shape=jax.ShapeDtypeStruct((N, C1), jnp.float32),
    )


@functools.lru_cache(maxsize=None)
def _make_stage1(N, NP, C1, CD):
    GB = 10
    RB = N // GB

    def body(degp, xw, hs1, dinv8):
        d = degp[0] + degp[1] + 1.0
        di = lax.rsqrt(d)
        dinv8[...] = di
        hs1[...] = xw[...] * di[:, 0:1]

    return pl.pallas_call(
        body,
        grid=(GB,),
        in_specs=[
            pl.BlockSpec((2, RB, CD), lambda i: (0, i, 0)),
            pl.BlockSpec((RB, C1), lambda i: (i, 0)),
        ],
        out_specs=[
            pl.BlockSpec((RB, C1), lambda i: (i, 0)),
            pl.BlockSpec((RB, CD), lambda i: (i, 0)),
        ],
        out_shape=[
            jax.ShapeDtypeStruct((N, C1), jnp.float32),
            jax.ShapeDtypeStruct((N, CD), jnp.float32),
        ],
    )


@functools.lru_cache(maxsize=None)
def _make_stage2(N, NP, C1, C2, CD):
    GB = 10
    RB = N // GB

    def body(acc1, hs1, dinv8, b1, w2, hs2):
        di = dinv8[:, 0:1]
        h = jnp.maximum((acc1[0] + acc1[1] + hs1[...]) * di + b1[...], 0.0)
        hs2[...] = (
            jnp.dot(h, w2[...], preferred_element_type=jnp.float32) * di
        )

    return pl.pallas_call(
        body,
        grid=(GB,),
        in_specs=[
            pl.BlockSpec((2, RB, C1), lambda i: (0, i, 0)),
            pl.BlockSpec((RB, C1), lambda i: (i, 0)),
            pl.BlockSpec((RB, CD), lambda i: (i, 0)),
            pl.BlockSpec((1, C1), lambda i: (0, 0)),
            pl.BlockSpec((C1, C2), lambda i: (0, 0)),
        ],
        out_specs=pl.BlockSpec((RB, C2), lambda i: (i, 0)),
        out_shape=jax.ShapeDtypeStruct((N, C2), jnp.float32),
    )


@functools.lru_cache(maxsize=None)
def _make_stage3(N, NP, C2, CD, OUT):
    GB = 10
    RB = N // GB

    def body(acc2, hs2, dinv8, b2, z):
        di = dinv8[:, 0:1]
        v = (acc2[0] + acc2[1] + hs2[...]) * di + b2[...]
        z[...] = v[:, :OUT]

    return pl.pallas_call(
        body,
        grid=(GB,),
        in_specs=[
            pl.BlockSpec((2, RB, C2), lambda i: (0, i, 0)),
            pl.BlockSpec((RB, C2), lambda i: (i, 0)),
            pl.BlockSpec((RB, CD), lambda i: (i, 0)),
            pl.BlockSpec((1, C2), lambda i: (0, 0)),
        ],
        out_specs=pl.BlockSpec((RB, OUT), lambda i: (i, 0)),
        out_shape=jax.ShapeDtypeStruct((N, OUT), jnp.float32),
    )


def kernel(x, edge_index, W1, b1, W2, b2):
    N, IN = x.shape
    HID = W1.shape[1]
    OUT = W2.shape[1]
    E = edge_index.shape[1]
    assert N % NS == 0

    C1 = 32
    C2 = 16
    CD = 8

    f32 = jnp.float32
    src = edge_index[0].astype(jnp.int32)
    dst = edge_index[1].astype(jnp.int32)
    NP = _pad_rows(N)
    EP = -(-E // (NW * K)) * (NW * K)
    if EP != E:
        pad = EP - E
        src = jnp.concatenate([src, jnp.zeros((pad,), jnp.int32)])
        pad_dst = N + jnp.arange(pad, dtype=jnp.int32) % (NP - N)
        dst = jnp.concatenate([dst, pad_dst])
    src2 = src.reshape(EP // K, K)
    dst2 = dst.reshape(EP // K, K)

    W1p = jnp.zeros((IN, C1), f32).at[:, :HID].set(W1)
    b1p = jnp.zeros((1, C1), f32).at[0, :HID].set(b1)
    W2p = jnp.zeros((C1, C2), f32).at[:HID, :OUT].set(W2)
    b2p = jnp.zeros((1, C2), f32).at[0, :OUT].set(b2)
    ROWS_T = _pad_rows(N) // NS
    zerosD = jnp.zeros((ROWS_T, CD), f32)
    zeros1 = jnp.zeros((ROWS_T, C1), f32)
    zeros2 = jnp.zeros((ROWS_T, C2), f32)
    onesD = jnp.ones((K, CD), f32)

    xw1 = _make_mm(N, IN, C1)(x, W1p)
    deg_parts = _make_seg_sum(N, EP, CD, False)(onesD, dst2, zerosD)
    hs1, dinv8 = _make_stage1(N, NP, C1, CD)(deg_parts, xw1)
    acc1 = _make_seg_sum(N, EP, C1, True)(hs1, src2, dst2, zeros1)
    hs2 = _make_stage2(N, NP, C1, C2, CD)(acc1, hs1, dinv8, b1p, W2p)
    acc2 = _make_seg_sum(N, EP, C2, True)(hs2, src2, dst2, zeros2)
    return _make_stage3(N, NP, C2, CD, OUT)(acc2, hs2, dinv8, b2p)


# speedup vs baseline: 39.6354x; 1.0329x over previous
import functools

import jax
import jax.numpy as jnp
from jax import lax
from jax.experimental import pallas as pl
from jax.experimental.pallas import tpu as pltpu
from jax.experimental.pallas import tpu_sc as plsc

NC = 2
NS = 16
NW = NC * NS
K = 128


def _pad_rows(N):
    return -(-(N + 1) // (NS * 8)) * (NS * 8)


@functools.lru_cache(maxsize=None)
def _make_seg_sum(N, EP, C, gather):
    CPT = EP // (NW * K)
    NB = 8
    D = 6
    LS = NB - D
    NP = _pad_rows(N)
    ROWS_T = NP // NS
    assert CPT >= D + LS + 1
    mesh = plsc.VectorSubcoreMesh(core_axis_name="c", subcore_axis_name="s")

    scratch = [
        pltpu.VMEM_SHARED((NP, C), jnp.float32),
        pltpu.VMEM((CPT, K), jnp.int32),
        pltpu.VMEM((NB, K, C), jnp.float32),
        pltpu.SemaphoreType.DMA,
    ]
    if gather:
        scratch += [
            pltpu.VMEM((CPT, K), jnp.int32),
            pltpu.SemaphoreType.DMA,
        ]

    @functools.partial(
        pl.kernel,
        out_type=jax.ShapeDtypeStruct((NC, NP, C), jnp.float32),
        mesh=mesh,
        scratch_types=scratch,
        compiler_params=pltpu.CompilerParams(use_tc_tiling_on_sc=False),
    )
    def k(*refs):
        if gather:
            table, src2, dst2, zeros, out, acc, dstb, rows, ssem, srcb, gsem = refs
        else:
            ones, dst2, zeros, out, acc, dstb, rows, ssem = refs

        cid = lax.axis_index("c")
        sid = lax.axis_index("s")
        w = cid * NS + sid

        pltpu.sync_copy(dst2.at[pl.ds(w * CPT, CPT)], dstb)

        def s_issue(j, b):
            pltpu.async_copy(rows.at[b], acc.at[dstb.at[j]], ssem, add=True)

        def s_wait(j, b):
            pltpu.make_async_copy(rows.at[b], acc.at[dstb.at[j]], ssem).wait()

        if gather:
            pltpu.sync_copy(src2.at[pl.ds(w * CPT, CPT)], srcb)

            def g_issue(j, b):
                pltpu.async_copy(table.at[srcb.at[j]], rows.at[b], gsem)

            def g_wait(j, b):
                pltpu.make_async_copy(
                    table.at[srcb.at[j]], rows.at[b], gsem
                ).wait()

            for j in range(D):
                g_issue(j, j)
        else:
            pltpu.sync_copy(ones, rows.at[0])

        pltpu.sync_copy(zeros, acc.at[pl.ds(sid * ROWS_T, ROWS_T)])
        plsc.subcore_barrier()

        if gather:
            for i in range(LS):
                g_wait(i, i % NB)
                s_issue(i, i % NB)
                g_issue(i + D, (i + D) % NB)

            def body(i, carry):
                b = lax.rem(i, NB)
                g_wait(i, b)
                s_issue(i, b)
                s_wait(i - LS, lax.rem(i - LS, NB))
                g_issue(i + D, lax.rem(i + D, NB))
                return carry

            lax.fori_loop(LS, CPT - D, body, 0)

            for i in range(CPT - D, CPT):
                g_wait(i, i % NB)
                s_issue(i, i % NB)
            for i in range(CPT - D - LS, CPT):
                s_wait(i, i % NB)
        else:
            F = 8
            for j in range(F):
                s_issue(j, 0)

            def body(i, carry):
                s_wait(i - F, 0)
                s_issue(i, 0)
                return carry

            lax.fori_loop(F, CPT, body, 0)
            for i in range(CPT - F, CPT):
                s_wait(i, 0)

        plsc.subcore_barrier()
        pltpu.sync_copy(
            acc.at[pl.ds(sid * ROWS_T, ROWS_T)],
            out.at[cid, pl.ds(sid * ROWS_T, ROWS_T)],
        )

    return k


@functools.lru_cache(maxsize=None)
def _make_mm(N, IN, C1):
    GB = 10
    RB = N // GB

    def body(xr, w, xw):
        xw[...] = jnp.dot(xr[...], w[...], preferred_element_type=jnp.float32)

    return pl.pallas_call(
        body,
        grid=(GB,),
        in_specs=[
            pl.BlockSpec((RB, IN), lambda i: (i, 0)),
            pl.BlockSpec((IN, C1), lambda i: (0, 0)),
        ],
        out_specs=pl.BlockSpec((RB, C1), lambda i: (i, 0)),
        out_shape=jax.ShapeDtypeStruct((N, C1), jnp.float32),
    )


@functools.lru_cache(maxsize=None)
def _make_stage1(N, NP, C1, CD):
    GB = 10
    RB = N // GB

    def body(degp, xw, hs1, dinv8):
        d = degp[0] + degp[1] + 1.0
        di = lax.rsqrt(d)
        dinv8[...] = di
        hs1[...] = xw[...] * di[:, 0:1]

    return pl.pallas_call(
        body,
        grid=(GB,),
        in_specs=[
            pl.BlockSpec((2, RB, CD), lambda i: (0, i, 0)),
            pl.BlockSpec((RB, C1), lambda i: (i, 0)),
        ],
        out_specs=[
            pl.BlockSpec((RB, C1), lambda i: (i, 0)),
            pl.BlockSpec((RB, CD), lambda i: (i, 0)),
        ],
        out_shape=[
            jax.ShapeDtypeStruct((N, C1), jnp.float32),
            jax.ShapeDtypeStruct((N, CD), jnp.float32),
        ],
    )


@functools.lru_cache(maxsize=None)
def _make_stage2(N, NP, C1, C2, CD):
    GB = 10
    RB = N // GB

    def body(acc1, hs1, dinv8, b1, w2, hs2):
        di = dinv8[:, 0:1]
        h = jnp.maximum((acc1[0] + acc1[1] + hs1[...]) * di + b1[...], 0.0)
        hs2[...] = (
            jnp.dot(h, w2[...], preferred_element_type=jnp.float32) * di
        )

    return pl.pallas_call(
        body,
        grid=(GB,),
        in_specs=[
            pl.BlockSpec((2, RB, C1), lambda i: (0, i, 0)),
            pl.BlockSpec((RB, C1), lambda i: (i, 0)),
            pl.BlockSpec((RB, CD), lambda i: (i, 0)),
            pl.BlockSpec((1, C1), lambda i: (0, 0)),
            pl.BlockSpec((C1, C2), lambda i: (0, 0)),
        ],
        out_specs=pl.BlockSpec((RB, C2), lambda i: (i, 0)),
        out_shape=jax.ShapeDtypeStruct((N, C2), jnp.float32),
    )


@functools.lru_cache(maxsize=None)
def _make_stage3(N, NP, C2, CD, OUT):
    GB = 10
    RB = N // GB

    def body(acc2, hs2, dinv8, b2, z):
        di = dinv8[:, 0:1]
        v = (acc2[0] + acc2[1] + hs2[...]) * di + b2[...]
        z[...] = v[:, :OUT]

    return pl.pallas_call(
        body,
        grid=(GB,),
        in_specs=[
            pl.BlockSpec((2, RB, C2), lambda i: (0, i, 0)),
            pl.BlockSpec((RB, C2), lambda i: (i, 0)),
            pl.BlockSpec((RB, CD), lambda i: (i, 0)),
            pl.BlockSpec((1, C2), lambda i: (0, 0)),
        ],
        out_specs=pl.BlockSpec((RB, OUT), lambda i: (i, 0)),
        out_shape=jax.ShapeDtypeStruct((N, OUT), jnp.float32),
    )


def kernel(x, edge_index, W1, b1, W2, b2):
    N, IN = x.shape
    HID = W1.shape[1]
    OUT = W2.shape[1]
    E = edge_index.shape[1]
    assert N % NS == 0

    C1 = 32
    C2 = 16
    CD = 8

    f32 = jnp.float32
    src = edge_index[0].astype(jnp.int32)
    dst = edge_index[1].astype(jnp.int32)
    NP = _pad_rows(N)
    EP = -(-E // (NW * K)) * (NW * K)
    if EP != E:
        pad = EP - E
        src = jnp.concatenate([src, jnp.zeros((pad,), jnp.int32)])
        pad_dst = N + jnp.arange(pad, dtype=jnp.int32) % (NP - N)
        dst = jnp.concatenate([dst, pad_dst])
    src2 = src.reshape(EP // K, K)
    dst2 = dst.reshape(EP // K, K)

    W1p = jnp.zeros((IN, C1), f32).at[:, :HID].set(W1)
    b1p = jnp.zeros((1, C1), f32).at[0, :HID].set(b1)
    W2p = jnp.zeros((C1, C2), f32).at[:HID, :OUT].set(W2)
    b2p = jnp.zeros((1, C2), f32).at[0, :OUT].set(b2)
    ROWS_T = _pad_rows(N) // NS
    zerosD = jnp.zeros((ROWS_T, CD), f32)
    zeros1 = jnp.zeros((ROWS_T, C1), f32)
    zeros2 = jnp.zeros((ROWS_T, C2), f32)
    onesD = jnp.ones((K, CD), f32)

    xw1 = _make_mm(N, IN, C1)(x, W1p)
    deg_parts = _make_seg_sum(N, EP, CD, False)(onesD, dst2, zerosD)
    hs1, dinv8 = _make_stage1(N, NP, C1, CD)(deg_parts, xw1)
    acc1 = _make_seg_sum(N, EP, C1, True)(hs1, src2, dst2, zeros1)
    hs2 = _make_stage2(N, NP, C1, C2, CD)(acc1, hs1, dinv8, b1p, W2p)
    acc2 = _make_seg_sum(N, EP, C2, True)(hs2, src2, dst2, zeros2)
    return _make_stage3(N, NP, C2, CD, OUT)(acc2, hs2, dinv8, b2p)


# speedup vs baseline: 42.0444x; 1.0608x over previous
import functools

import jax
import jax.numpy as jnp
from jax import lax
from jax.experimental import pallas as pl
from jax.experimental.pallas import tpu as pltpu
from jax.experimental.pallas import tpu_sc as plsc

NC = 2
NS = 16
NW = NC * NS
K = 128


def _pad_rows(N):
    return -(-(N + 1) // (NS * 8)) * (NS * 8)


@functools.lru_cache(maxsize=None)
def _make_seg_sum(N, EP, C, gather, CPT0):
    CPT = EP // (NW * K)
    CPT1 = 2 * CPT - CPT0
    NB = 8
    D = 6
    LS = NB - D
    NP = _pad_rows(N)
    ROWS_T = NP // NS
    assert CPT1 >= D + LS + 1 and CPT0 >= CPT1
    mesh = plsc.VectorSubcoreMesh(core_axis_name="c", subcore_axis_name="s")

    scratch = [
        pltpu.VMEM_SHARED((NP, C), jnp.float32),
        pltpu.VMEM((CPT0, K), jnp.int32),
        pltpu.VMEM((NB, K, C), jnp.float32),
        pltpu.SemaphoreType.DMA,
    ]
    if gather:
        scratch += [
            pltpu.VMEM((CPT0, K), jnp.int32),
            pltpu.SemaphoreType.DMA,
        ]

    @functools.partial(
        pl.kernel,
        out_type=jax.ShapeDtypeStruct((NC, NP, C), jnp.float32),
        mesh=mesh,
        scratch_types=scratch,
        compiler_params=pltpu.CompilerParams(use_tc_tiling_on_sc=False),
    )
    def k(*refs):
        if gather:
            table, src2, dst2, zeros, out, acc, dstb, rows, ssem, srcb, gsem = refs
        else:
            ones, dst2, zeros, out, acc, dstb, rows, ssem = refs

        cid = lax.axis_index("c")
        sid = lax.axis_index("s")
        cnt = CPT0 - cid * (CPT0 - CPT1)
        cbase = cid * (NS * CPT0) + sid * cnt

        pltpu.sync_copy(dst2.at[pl.ds(cbase, CPT0)], dstb)

        def s_issue(j, b):
            pltpu.async_copy(rows.at[b], acc.at[dstb.at[j]], ssem, add=True)

        def s_wait(j, b):
            pltpu.make_async_copy(rows.at[b], acc.at[dstb.at[j]], ssem).wait()

        if gather:
            pltpu.sync_copy(src2.at[pl.ds(cbase, CPT0)], srcb)

            def g_issue(j, b):
                pltpu.async_copy(table.at[srcb.at[j]], rows.at[b], gsem)

            def g_wait(j, b):
                pltpu.make_async_copy(
                    table.at[srcb.at[j]], rows.at[b], gsem
                ).wait()

            for j in range(D):
                g_issue(j, j)
        else:
            pltpu.sync_copy(ones, rows.at[0])

        pltpu.sync_copy(zeros, acc.at[pl.ds(sid * ROWS_T, ROWS_T)])
        plsc.subcore_barrier()

        if gather:
            for i in range(LS):
                g_wait(i, i % NB)
                s_issue(i, i % NB)
                g_issue(i + D, (i + D) % NB)

            def body(i, carry):
                b = lax.rem(i, NB)
                g_wait(i, b)
                s_issue(i, b)
                s_wait(i - LS, lax.rem(i - LS, NB))
                g_issue(i + D, lax.rem(i + D, NB))
                return carry

            lax.fori_loop(LS, cnt - D, body, 0)

            def tail(i, carry):
                b = lax.rem(i, NB)
                g_wait(i, b)
                s_issue(i, b)
                return carry

            lax.fori_loop(cnt - D, cnt, tail, 0)

            def drain(i, carry):
                s_wait(i, lax.rem(i, NB))
                return carry

            lax.fori_loop(cnt - D - LS, cnt, drain, 0)
        else:
            F = 8
            for j in range(F):
                s_issue(j, 0)

            def body(i, carry):
                s_wait(i - F, 0)
                s_issue(i, 0)
                return carry

            lax.fori_loop(F, cnt, body, 0)

            def drain(i, carry):
                s_wait(i, 0)
                return carry

            lax.fori_loop(cnt - F, cnt, drain, 0)

        plsc.subcore_barrier()
        pltpu.sync_copy(
            acc.at[pl.ds(sid * ROWS_T, ROWS_T)],
            out.at[cid, pl.ds(sid * ROWS_T, ROWS_T)],
        )

    return k


@functools.lru_cache(maxsize=None)
def _make_mm(N, IN, C1):
    GB = 10
    RB = N // GB

    def body(xr, w, xw):
        xw[...] = jnp.dot(xr[...], w[...], preferred_element_type=jnp.float32)

    return pl.pallas_call(
        body,
        grid=(GB,),
        in_specs=[
            pl.BlockSpec((RB, IN), lambda i: (i, 0)),
            pl.BlockSpec((IN, C1), lambda i: (0, 0)),
        ],
        out_specs=pl.BlockSpec((RB, C1), lambda i: (i, 0)),
        out_shape=jax.ShapeDtypeStruct((N, C1), jnp.float32),
    )


@functools.lru_cache(maxsize=None)
def _make_stage1(N, NP, C1, CD):
    GB = 10
    RB = N // GB

    def body(degp, xw, hs1, dinv8):
        d = degp[0] + degp[1] + 1.0
        di = lax.rsqrt(d)
        dinv8[...] = di
        hs1[...] = xw[...] * di[:, 0:1]

    return pl.pallas_call(
        body,
        grid=(GB,),
        in_specs=[
            pl.BlockSpec((2, RB, CD), lambda i: (0, i, 0)),
            pl.BlockSpec((RB, C1), lambda i: (i, 0)),
        ],
        out_specs=[
            pl.BlockSpec((RB, C1), lambda i: (i, 0)),
            pl.BlockSpec((RB, CD), lambda i: (i, 0)),
        ],
        out_shape=[
            jax.ShapeDtypeStruct((N, C1), jnp.float32),
            jax.ShapeDtypeStruct((N, CD), jnp.float32),
        ],
    )


@functools.lru_cache(maxsize=None)
def _make_stage2(N, NP, C1, C2, CD):
    GB = 10
    RB = N // GB

    def body(acc1, hs1, dinv8, b1, w2, hs2):
        di = dinv8[:, 0:1]
        h = jnp.maximum((acc1[0] + acc1[1] + hs1[...]) * di + b1[...], 0.0)
        hs2[...] = (
            jnp.dot(h, w2[...], preferred_element_type=jnp.float32) * di
        )

    return pl.pallas_call(
        body,
        grid=(GB,),
        in_specs=[
            pl.BlockSpec((2, RB, C1), lambda i: (0, i, 0)),
            pl.BlockSpec((RB, C1), lambda i: (i, 0)),
            pl.BlockSpec((RB, CD), lambda i: (i, 0)),
            pl.BlockSpec((1, C1), lambda i: (0, 0)),
            pl.BlockSpec((C1, C2), lambda i: (0, 0)),
        ],
        out_specs=pl.BlockSpec((RB, C2), lambda i: (i, 0)),
        out_shape=jax.ShapeDtypeStruct((N, C2), jnp.float32),
    )


@functools.lru_cache(maxsize=None)
def _make_stage3(N, NP, C2, CD, OUT):
    GB = 10
    RB = N // GB

    def body(acc2, hs2, dinv8, b2, z):
        di = dinv8[:, 0:1]
        v = (acc2[0] + acc2[1] + hs2[...]) * di + b2[...]
        z[...] = v[:, :OUT]

    return pl.pallas_call(
        body,
        grid=(GB,),
        in_specs=[
            pl.BlockSpec((2, RB, C2), lambda i: (0, i, 0)),
            pl.BlockSpec((RB, C2), lambda i: (i, 0)),
            pl.BlockSpec((RB, CD), lambda i: (i, 0)),
            pl.BlockSpec((1, C2), lambda i: (0, 0)),
        ],
        out_specs=pl.BlockSpec((RB, OUT), lambda i: (i, 0)),
        out_shape=jax.ShapeDtypeStruct((N, OUT), jnp.float32),
    )


def kernel(x, edge_index, W1, b1, W2, b2):
    N, IN = x.shape
    HID = W1.shape[1]
    OUT = W2.shape[1]
    E = edge_index.shape[1]
    assert N % NS == 0

    C1 = 32
    C2 = 16
    CD = 8

    f32 = jnp.float32
    src = edge_index[0].astype(jnp.int32)
    dst = edge_index[1].astype(jnp.int32)
    NP = _pad_rows(N)
    EP = -(-E // (NW * K)) * (NW * K)
    CPT = EP // (NW * K)
    CPT0_L1 = min(2 * CPT - 9, int(round(2 * CPT * 0.72)))
    CPT0_L2 = min(2 * CPT - 9, int(round(2 * CPT * 0.58)))
    spare = max(CPT0_L1, CPT0_L2) - CPT
    pad = EP - E + spare * K
    if pad:
        src = jnp.concatenate([src, jnp.zeros((pad,), jnp.int32)])
        pad_dst = N + jnp.arange(pad, dtype=jnp.int32) % (NP - N)
        dst = jnp.concatenate([dst, pad_dst])
    src2 = src.reshape(-1, K)
    dst2 = dst.reshape(-1, K)

    W1p = jnp.zeros((IN, C1), f32).at[:, :HID].set(W1)
    b1p = jnp.zeros((1, C1), f32).at[0, :HID].set(b1)
    W2p = jnp.zeros((C1, C2), f32).at[:HID, :OUT].set(W2)
    b2p = jnp.zeros((1, C2), f32).at[0, :OUT].set(b2)
    ROWS_T = _pad_rows(N) // NS
    zerosD = jnp.zeros((ROWS_T, CD), f32)
    zeros1 = jnp.zeros((ROWS_T, C1), f32)
    zeros2 = jnp.zeros((ROWS_T, C2), f32)
    onesD = jnp.ones((K, CD), f32)

    xw1 = _make_mm(N, IN, C1)(x, W1p)
    deg_parts = _make_seg_sum(N, EP, CD, False, CPT)(onesD, dst2, zerosD)
    hs1, dinv8 = _make_stage1(N, NP, C1, CD)(deg_parts, xw1)
    acc1 = _make_seg_sum(N, EP, C1, True, CPT0_L1)(hs1, src2, dst2, zeros1)
    hs2 = _make_stage2(N, NP, C1, C2, CD)(acc1, hs1, dinv8, b1p, W2p)
    acc2 = _make_seg_sum(N, EP, C2, True, CPT0_L2)(hs2, src2, dst2, zeros2)
    return _make_stage3(N, NP, C2, CD, OUT)(acc2, hs2, dinv8, b2p)


# speedup vs baseline: 42.1137x; 1.0016x over previous
import functools

import jax
import jax.numpy as jnp
from jax import lax
from jax.experimental import pallas as pl
from jax.experimental.pallas import tpu as pltpu
from jax.experimental.pallas import tpu_sc as plsc

NC = 2
NS = 16
NW = NC * NS
K = 128


def _pad_rows(N):
    return -(-(N + 1) // (NS * 8)) * (NS * 8)


@functools.lru_cache(maxsize=None)
def _make_seg_sum(N, EP, C, gather, CPT0):
    CPT = EP // (NW * K)
    CPT1 = 2 * CPT - CPT0
    NB = 16
    D = 13
    LS = NB - D
    NP = _pad_rows(N)
    ROWS_T = NP // NS
    assert CPT1 >= D + LS + 1 and CPT0 >= CPT1
    mesh = plsc.VectorSubcoreMesh(core_axis_name="c", subcore_axis_name="s")

    scratch = [
        pltpu.VMEM_SHARED((NP, C), jnp.float32),
        pltpu.VMEM((CPT0, K), jnp.int32),
        pltpu.VMEM((NB, K, C), jnp.float32),
        pltpu.SemaphoreType.DMA,
    ]
    if gather:
        scratch += [
            pltpu.VMEM((CPT0, K), jnp.int32),
            pltpu.SemaphoreType.DMA,
        ]

    @functools.partial(
        pl.kernel,
        out_type=jax.ShapeDtypeStruct((NC, NP, C), jnp.float32),
        mesh=mesh,
        scratch_types=scratch,
        compiler_params=pltpu.CompilerParams(use_tc_tiling_on_sc=False),
    )
    def k(*refs):
        if gather:
            table, src2, dst2, zeros, out, acc, dstb, rows, ssem, srcb, gsem = refs
        else:
            ones, dst2, zeros, out, acc, dstb, rows, ssem = refs

        cid = lax.axis_index("c")
        sid = lax.axis_index("s")
        cnt = CPT0 - cid * (CPT0 - CPT1)
        cbase = cid * (NS * CPT0) + sid * cnt

        pltpu.sync_copy(dst2.at[pl.ds(cbase, CPT0)], dstb)

        def s_issue(j, b):
            pltpu.async_copy(rows.at[b], acc.at[dstb.at[j]], ssem, add=True)

        def s_wait(j, b):
            pltpu.make_async_copy(rows.at[b], acc.at[dstb.at[j]], ssem).wait()

        if gather:
            pltpu.sync_copy(src2.at[pl.ds(cbase, CPT0)], srcb)

            def g_issue(j, b):
                pltpu.async_copy(table.at[srcb.at[j]], rows.at[b], gsem)

            def g_wait(j, b):
                pltpu.make_async_copy(
                    table.at[srcb.at[j]], rows.at[b], gsem
                ).wait()

            for j in range(D):
                g_issue(j, j)
        else:
            pltpu.sync_copy(ones, rows.at[0])

        pltpu.sync_copy(zeros, acc.at[pl.ds(sid * ROWS_T, ROWS_T)])
        plsc.subcore_barrier()

        if gather:
            for i in range(LS):
                g_wait(i, i % NB)
                s_issue(i, i % NB)
                g_issue(i + D, (i + D) % NB)

            def body(i, carry):
                b = lax.rem(i, NB)
                g_wait(i, b)
                s_issue(i, b)
                s_wait(i - LS, lax.rem(i - LS, NB))
                g_issue(i + D, lax.rem(i + D, NB))
                return carry

            lax.fori_loop(LS, cnt - D, body, 0)

            def tail(i, carry):
                b = lax.rem(i, NB)
                g_wait(i, b)
                s_issue(i, b)
                return carry

            lax.fori_loop(cnt - D, cnt, tail, 0)

            def drain(i, carry):
                s_wait(i, lax.rem(i, NB))
                return carry

            lax.fori_loop(cnt - D - LS, cnt, drain, 0)
        else:
            F = 14
            for j in range(F):
                s_issue(j, 0)

            def body(i, carry):
                s_wait(i - F, 0)
                s_issue(i, 0)
                return carry

            lax.fori_loop(F, cnt, body, 0)

            def drain(i, carry):
                s_wait(i, 0)
                return carry

            lax.fori_loop(cnt - F, cnt, drain, 0)

        plsc.subcore_barrier()
        pltpu.sync_copy(
            acc.at[pl.ds(sid * ROWS_T, ROWS_T)],
            out.at[cid, pl.ds(sid * ROWS_T, ROWS_T)],
        )

    return k


@functools.lru_cache(maxsize=None)
def _make_mm(N, IN, C1):
    GB = 10
    RB = N // GB

    def body(xr, w, xw):
        xw[...] = jnp.dot(xr[...], w[...], preferred_element_type=jnp.float32)

    return pl.pallas_call(
        body,
        grid=(GB,),
        in_specs=[
            pl.BlockSpec((RB, IN), lambda i: (i, 0)),
            pl.BlockSpec((IN, C1), lambda i: (0, 0)),
        ],
        out_specs=pl.BlockSpec((RB, C1), lambda i: (i, 0)),
        out_shape=jax.ShapeDtypeStruct((N, C1), jnp.float32),
    )


@functools.lru_cache(maxsize=None)
def _make_stage1(N, NP, C1, CD):
    GB = 10
    RB = N // GB

    def body(degp, xw, hs1, dinv8):
        d = degp[0] + degp[1] + 1.0
        di = lax.rsqrt(d)
        dinv8[...] = di
        hs1[...] = xw[...] * di[:, 0:1]

    return pl.pallas_call(
        body,
        grid=(GB,),
        in_specs=[
            pl.BlockSpec((2, RB, CD), lambda i: (0, i, 0)),
            pl.BlockSpec((RB, C1), lambda i: (i, 0)),
        ],
        out_specs=[
            pl.BlockSpec((RB, C1), lambda i: (i, 0)),
            pl.BlockSpec((RB, CD), lambda i: (i, 0)),
        ],
        out_shape=[
            jax.ShapeDtypeStruct((N, C1), jnp.float32),
            jax.ShapeDtypeStruct((N, CD), jnp.float32),
        ],
    )


@functools.lru_cache(maxsize=None)
def _make_stage2(N, NP, C1, C2, CD):
    GB = 10
    RB = N // GB

    def body(acc1, hs1, dinv8, b1, w2, hs2):
        di = dinv8[:, 0:1]
        h = jnp.maximum((acc1[0] + acc1[1] + hs1[...]) * di + b1[...], 0.0)
        hs2[...] = (
            jnp.dot(h, w2[...], preferred_element_type=jnp.float32) * di
        )

    return pl.pallas_call(
        body,
        grid=(GB,),
        in_specs=[
            pl.BlockSpec((2, RB, C1), lambda i: (0, i, 0)),
            pl.BlockSpec((RB, C1), lambda i: (i, 0)),
            pl.BlockSpec((RB, CD), lambda i: (i, 0)),
            pl.BlockSpec((1, C1), lambda i: (0, 0)),
            pl.BlockSpec((C1, C2), lambda i: (0, 0)),
        ],
        out_specs=pl.BlockSpec((RB, C2), lambda i: (i, 0)),
        out_shape=jax.ShapeDtypeStruct((N, C2), jnp.float32),
    )


@functools.lru_cache(maxsize=None)
def _make_stage3(N, NP, C2, CD, OUT):
    GB = 10
    RB = N // GB

    def body(acc2, hs2, dinv8, b2, z):
        di = dinv8[:, 0:1]
        v = (acc2[0] + acc2[1] + hs2[...]) * di + b2[...]
        z[...] = v[:, :OUT]

    return pl.pallas_call(
        body,
        grid=(GB,),
        in_specs=[
            pl.BlockSpec((2, RB, C2), lambda i: (0, i, 0)),
            pl.BlockSpec((RB, C2), lambda i: (i, 0)),
            pl.BlockSpec((RB, CD), lambda i: (i, 0)),
            pl.BlockSpec((1, C2), lambda i: (0, 0)),
        ],
        out_specs=pl.BlockSpec((RB, OUT), lambda i: (i, 0)),
        out_shape=jax.ShapeDtypeStruct((N, OUT), jnp.float32),
    )


def kernel(x, edge_index, W1, b1, W2, b2):
    N, IN = x.shape
    HID = W1.shape[1]
    OUT = W2.shape[1]
    E = edge_index.shape[1]
    assert N % NS == 0

    C1 = 32
    C2 = 16
    CD = 8

    f32 = jnp.float32
    src = edge_index[0].astype(jnp.int32)
    dst = edge_index[1].astype(jnp.int32)
    NP = _pad_rows(N)
    EP = -(-E // (NW * K)) * (NW * K)
    CPT = EP // (NW * K)
    CPT0_L1 = min(2 * CPT - 9, int(round(2 * CPT * 0.72)))
    CPT0_L2 = min(2 * CPT - 9, int(round(2 * CPT * 0.58)))
    spare = max(CPT0_L1, CPT0_L2) - CPT
    pad = EP - E + spare * K
    if pad:
        src = jnp.concatenate([src, jnp.zeros((pad,), jnp.int32)])
        pad_dst = N + jnp.arange(pad, dtype=jnp.int32) % (NP - N)
        dst = jnp.concatenate([dst, pad_dst])
    src2 = src.reshape(-1, K)
    dst2 = dst.reshape(-1, K)

    W1p = jnp.zeros((IN, C1), f32).at[:, :HID].set(W1)
    b1p = jnp.zeros((1, C1), f32).at[0, :HID].set(b1)
    W2p = jnp.zeros((C1, C2), f32).at[:HID, :OUT].set(W2)
    b2p = jnp.zeros((1, C2), f32).at[0, :OUT].set(b2)
    ROWS_T = _pad_rows(N) // NS
    zerosD = jnp.zeros((ROWS_T, CD), f32)
    zeros1 = jnp.zeros((ROWS_T, C1), f32)
    zeros2 = jnp.zeros((ROWS_T, C2), f32)
    onesD = jnp.ones((K, CD), f32)

    xw1 = _make_mm(N, IN, C1)(x, W1p)
    deg_parts = _make_seg_sum(N, EP, CD, False, CPT)(onesD, dst2, zerosD)
    hs1, dinv8 = _make_stage1(N, NP, C1, CD)(deg_parts, xw1)
    acc1 = _make_seg_sum(N, EP, C1, True, CPT0_L1)(hs1, src2, dst2, zeros1)
    hs2 = _make_stage2(N, NP, C1, C2, CD)(acc1, hs1, dinv8, b1p, W2p)
    acc2 = _make_seg_sum(N, EP, C2, True, CPT0_L2)(hs2, src2, dst2, zeros2)
    return _make_stage3(N, NP, C2, CD, OUT)(acc2, hs2, dinv8, b2p)


# speedup vs baseline: 42.6084x; 1.0117x over previous
import functools

import jax
import jax.numpy as jnp
from jax import lax
from jax.experimental import pallas as pl
from jax.experimental.pallas import tpu as pltpu
from jax.experimental.pallas import tpu_sc as plsc

NC = 2
NS = 16
NW = NC * NS
K = 128


def _pad_rows(N):
    return -(-(N + 1) // (NS * 8)) * (NS * 8)


@functools.lru_cache(maxsize=None)
def _make_seg_sum(N, EP, C, gather, CPT0):
    CPT = EP // (NW * K)
    CPT1 = 2 * CPT - CPT0
    NB = 16
    D = 13
    LS = NB - D
    NP = _pad_rows(N)
    ROWS_T = NP // NS
    assert CPT1 >= D + LS + 1 and CPT0 >= CPT1
    mesh = plsc.VectorSubcoreMesh(core_axis_name="c", subcore_axis_name="s")

    scratch = [
        pltpu.VMEM_SHARED((NP, C), jnp.float32),
        pltpu.VMEM((CPT0, K), jnp.int32),
        pltpu.VMEM((NB, K, C), jnp.float32),
        pltpu.SemaphoreType.DMA,
    ]
    if gather:
        scratch += [
            pltpu.VMEM((CPT0, K), jnp.int32),
            pltpu.SemaphoreType.DMA,
            pltpu.VMEM((K, C), jnp.float32),
        ]

    @functools.partial(
        pl.kernel,
        out_type=jax.ShapeDtypeStruct((NC, NP, C), jnp.float32),
        mesh=mesh,
        scratch_types=scratch,
        compiler_params=pltpu.CompilerParams(use_tc_tiling_on_sc=False),
    )
    def k(*refs):
        if gather:
            table, src2, dst2, out, acc, dstb, rows, ssem, srcb, gsem, zb = refs
        else:
            ones, dst2, zeros, out, acc, dstb, rows, ssem = refs

        cid = lax.axis_index("c")
        sid = lax.axis_index("s")
        cnt = CPT0 - cid * (CPT0 - CPT1)
        cbase = cid * (NS * CPT0) + sid * cnt

        pltpu.sync_copy(dst2.at[pl.ds(cbase, CPT0)], dstb)

        def s_issue(j, b):
            pltpu.async_copy(rows.at[b], acc.at[dstb.at[j]], ssem, add=True)

        def s_wait(j, b):
            pltpu.make_async_copy(rows.at[b], acc.at[dstb.at[j]], ssem).wait()

        if gather:
            pltpu.sync_copy(src2.at[pl.ds(cbase, CPT0)], srcb)

            def g_issue(j, b):
                pltpu.async_copy(table.at[srcb.at[j]], rows.at[b], gsem)

            def g_wait(j, b):
                pltpu.make_async_copy(
                    table.at[srcb.at[j]], rows.at[b], gsem
                ).wait()

            CV = C // 16

            def zloop(i, carry):
                zb[i // CV, pl.ds((i % CV) * 16, 16)] = jnp.zeros(
                    (16,), jnp.float32
                )
                return carry

            lax.fori_loop(0, K * CV, zloop, 0)
            for t in range(ROWS_T // K):
                pltpu.sync_copy(
                    zb, acc.at[pl.ds(sid * ROWS_T + t * K, K)]
                )

            for j in range(D):
                g_issue(j, j)
        else:
            pltpu.sync_copy(ones, rows.at[0])
            pltpu.sync_copy(zeros, acc.at[pl.ds(sid * ROWS_T, ROWS_T)])

        plsc.subcore_barrier()

        if gather:
            for i in range(LS):
                g_wait(i, i % NB)
                s_issue(i, i % NB)
                g_issue(i + D, (i + D) % NB)

            def body(i, carry):
                b = lax.rem(i, NB)
                g_wait(i, b)
                s_issue(i, b)
                s_wait(i - LS, lax.rem(i - LS, NB))
                g_issue(i + D, lax.rem(i + D, NB))
                return carry

            lax.fori_loop(LS, cnt - D, body, 0)

            def tail(i, carry):
                b = lax.rem(i, NB)
                g_wait(i, b)
                s_issue(i, b)
                return carry

            lax.fori_loop(cnt - D, cnt, tail, 0)

            def drain(i, carry):
                s_wait(i, lax.rem(i, NB))
                return carry

            lax.fori_loop(cnt - D - LS, cnt, drain, 0)
        else:
            F = 14
            for j in range(F):
                s_issue(j, 0)

            def body(i, carry):
                s_wait(i - F, 0)
                s_issue(i, 0)
                return carry

            lax.fori_loop(F, cnt, body, 0)

            def drain(i, carry):
                s_wait(i, 0)
                return carry

            lax.fori_loop(cnt - F, cnt, drain, 0)

        plsc.subcore_barrier()
        pltpu.sync_copy(
            acc.at[pl.ds(sid * ROWS_T, ROWS_T)],
            out.at[cid, pl.ds(sid * ROWS_T, ROWS_T)],
        )

    return k


@functools.lru_cache(maxsize=None)
def _make_mm(N, IN, C1):
    GB = 10
    RB = N // GB

    def body(xr, w, xw):
        xw[...] = jnp.dot(xr[...], w[...], preferred_element_type=jnp.float32)

    return pl.pallas_call(
        body,
        grid=(GB,),
        in_specs=[
            pl.BlockSpec((RB, IN), lambda i: (i, 0)),
            pl.BlockSpec((IN, C1), lambda i: (0, 0)),
        ],
        out_specs=pl.BlockSpec((RB, C1), lambda i: (i, 0)),
        out_shape=jax.ShapeDtypeStruct((N, C1), jnp.float32),
    )


@functools.lru_cache(maxsize=None)
def _make_stage1(N, NP, C1, CD):
    GB = 10
    RB = N // GB

    def body(degp, xw, hs1, dinv8):
        d = degp[0] + degp[1] + 1.0
        di = lax.rsqrt(d)
        dinv8[...] = di
        hs1[...] = xw[...] * di[:, 0:1]

    return pl.pallas_call(
        body,
        grid=(GB,),
        in_specs=[
            pl.BlockSpec((2, RB, CD), lambda i: (0, i, 0)),
            pl.BlockSpec((RB, C1), lambda i: (i, 0)),
        ],
        out_specs=[
            pl.BlockSpec((RB, C1), lambda i: (i, 0)),
            pl.BlockSpec((RB, CD), lambda i: (i, 0)),
        ],
        out_shape=[
            jax.ShapeDtypeStruct((N, C1), jnp.float32),
            jax.ShapeDtypeStruct((N, CD), jnp.float32),
        ],
    )


@functools.lru_cache(maxsize=None)
def _make_stage2(N, NP, C1, C2, CD):
    GB = 10
    RB = N // GB

    def body(acc1, hs1, dinv8, b1, w2, hs2):
        di = dinv8[:, 0:1]
        h = jnp.maximum((acc1[0] + acc1[1] + hs1[...]) * di + b1[...], 0.0)
        hs2[...] = (
            jnp.dot(h, w2[...], preferred_element_type=jnp.float32) * di
        )

    return pl.pallas_call(
        body,
        grid=(GB,),
        in_specs=[
            pl.BlockSpec((2, RB, C1), lambda i: (0, i, 0)),
            pl.BlockSpec((RB, C1), lambda i: (i, 0)),
            pl.BlockSpec((RB, CD), lambda i: (i, 0)),
            pl.BlockSpec((1, C1), lambda i: (0, 0)),
            pl.BlockSpec((C1, C2), lambda i: (0, 0)),
        ],
        out_specs=pl.BlockSpec((RB, C2), lambda i: (i, 0)),
        out_shape=jax.ShapeDtypeStruct((N, C2), jnp.float32),
    )


@functools.lru_cache(maxsize=None)
def _make_stage3(N, NP, C2, CD, OUT):
    GB = 10
    RB = N // GB

    def body(acc2, hs2, dinv8, b2, z):
        di = dinv8[:, 0:1]
        v = (acc2[0] + acc2[1] + hs2[...]) * di + b2[...]
        z[...] = v[:, :OUT]

    return pl.pallas_call(
        body,
        grid=(GB,),
        in_specs=[
            pl.BlockSpec((2, RB, C2), lambda i: (0, i, 0)),
            pl.BlockSpec((RB, C2), lambda i: (i, 0)),
            pl.BlockSpec((RB, CD), lambda i: (i, 0)),
            pl.BlockSpec((1, C2), lambda i: (0, 0)),
        ],
        out_specs=pl.BlockSpec((RB, OUT), lambda i: (i, 0)),
        out_shape=jax.ShapeDtypeStruct((N, OUT), jnp.float32),
    )


def kernel(x, edge_index, W1, b1, W2, b2):
    N, IN = x.shape
    HID = W1.shape[1]
    OUT = W2.shape[1]
    E = edge_index.shape[1]
    assert N % NS == 0

    C1 = 32
    C2 = 16
    CD = 8

    f32 = jnp.float32
    src = edge_index[0].astype(jnp.int32)
    dst = edge_index[1].astype(jnp.int32)
    NP = _pad_rows(N)
    EP = -(-E // (NW * K)) * (NW * K)
    CPT = EP // (NW * K)
    CPT0_L1 = min(2 * CPT - 9, int(round(2 * CPT * 0.72)))
    CPT0_L2 = min(2 * CPT - 9, int(round(2 * CPT * 0.58)))
    spare = max(CPT0_L1, CPT0_L2) - CPT
    pad = EP - E + spare * K
    if pad:
        src = jnp.concatenate([src, jnp.zeros((pad,), jnp.int32)])
        pad_dst = N + jnp.arange(pad, dtype=jnp.int32) % (NP - N)
        dst = jnp.concatenate([dst, pad_dst])
    src2 = src.reshape(-1, K)
    dst2 = dst.reshape(-1, K)

    W1p = jnp.zeros((IN, C1), f32).at[:, :HID].set(W1)
    b1p = jnp.zeros((1, C1), f32).at[0, :HID].set(b1)
    W2p = jnp.zeros((C1, C2), f32).at[:HID, :OUT].set(W2)
    b2p = jnp.zeros((1, C2), f32).at[0, :OUT].set(b2)
    ROWS_T = _pad_rows(N) // NS
    zerosD = jnp.zeros((ROWS_T, CD), f32)
    onesD = jnp.ones((K, CD), f32)

    xw1 = _make_mm(N, IN, C1)(x, W1p)
    deg_parts = _make_seg_sum(N, EP, CD, False, CPT)(onesD, dst2, zerosD)
    hs1, dinv8 = _make_stage1(N, NP, C1, CD)(deg_parts, xw1)
    acc1 = _make_seg_sum(N, EP, C1, True, CPT0_L1)(hs1, src2, dst2)
    hs2 = _make_stage2(N, NP, C1, C2, CD)(acc1, hs1, dinv8, b1p, W2p)
    acc2 = _make_seg_sum(N, EP, C2, True, CPT0_L2)(hs2, src2, dst2)
    return _make_stage3(N, NP, C2, CD, OUT)(acc2, hs2, dinv8, b2p)
